# Initial kernel scaffold; baseline (speedup 1.0000x reference)
import jax
import jax.numpy as jnp
from jax.experimental import pallas as pl


def kernel(features, edge_index):
    raise NotImplementedError("write your pallas kernel here")


# speedup vs baseline: 15.9389x; 15.9389x over previous
import functools

import jax
import jax.numpy as jnp
from jax import lax
from jax.experimental import pallas as pl
from jax.experimental.pallas import tpu as pltpu
from jax.experimental.pallas import tpu_sc as plsc

L = 16


def _rsqrt16(v):
  i = plsc.bitcast(v, jnp.int32)
  y = plsc.bitcast(jnp.int32(0x5F3759DF) - (i >> 1), jnp.float32)
  for _ in range(3):
    y = y * (1.5 - 0.5 * v * y * y)
  return jnp.where(v > 0.5, y, jnp.zeros_like(y))


def _make_kernel(B, N, C, E, interpret=False):
  NSC = 2
  NT = 16
  NW = NSC * NT
  CK = 8
  assert C == CK * L

  NPT = 8 * (-(-(-(-N // NW)) // 8))
  STG = 64 * (-(-N // (64 * NT)))
  assert STG * NT >= N and N % 8 == 0 and N > NPT
  HR = L * (-(-N // (128 * L)))
  HP = HR * 128
  CHUNK = 2000
  EPT = E // NT
  assert EPT % CHUNK == 0 and E % CHUNK == 0 and CHUNK % L == 0
  assert HR <= 128 and HR % L == 0
  CAP = ((E // NW + 1024) // 128) * 128
  G = 128

  mesh = plsc.VectorSubcoreMesh(core_axis_name="c", subcore_axis_name="s")

  @functools.partial(
      pl.kernel,
      out_type=jax.ShapeDtypeStruct((NW * L,), jnp.float32),
      mesh=mesh,
      interpret=interpret,
      compiler_params=pltpu.CompilerParams(needs_layout_passes=False),
      scratch_types=dict(
          hist=pltpu.VMEM((HR, 128), jnp.float32),
          rowb=pltpu.VMEM((CAP + G,), jnp.int32),
          cwb=pltpu.VMEM((CAP + G,), jnp.int32),
          acc=pltpu.VMEM((NPT, C), jnp.float32),
          gbuf=pltpu.VMEM((G, C), jnp.float32),
          gidx=pltpu.VMEM((G,), jnp.int32),
          hidx=pltpu.VMEM((HR,), jnp.int32),
          rowck=pltpu.VMEM((CHUNK,), jnp.int32),
          colck=pltpu.VMEM((CHUNK,), jnp.int32),
          zbuf=pltpu.VMEM((8, C), jnp.float32),
          sbuf=pltpu.VMEM((L,), jnp.float32),
          sh_hist=pltpu.VMEM_SHARED((HR, 128), jnp.float32),
      ),
  )
  def kern(features, erow, ecol, out, hist, rowb, cwb, acc, gbuf,
           gidx, hidx, rowck, colck, zbuf, sbuf, sh_hist):
    s = lax.axis_index("s")
    c = lax.axis_index("c")
    w = c * NT + s
    zero16 = jnp.zeros((L,), jnp.float32)
    iota16 = lax.iota(jnp.int32, L)

    def z_hist(r, _):
      for k in range(CK):
        hist[r, pl.ds(k * L, L)] = zero16
      return 0
    lax.fori_loop(0, HR, z_hist, 0)

    def z_edges(i, _):
      rowb[pl.ds(i * L, L)] = jnp.zeros((L,), jnp.int32)
      cwb[pl.ds(i * L, L)] = jnp.zeros((L,), jnp.int32)
      return 0
    lax.fori_loop(0, (CAP + G) // L, z_edges, 0)

    for k in range(CK):
      for r in range(8):
        zbuf[r, pl.ds(k * L, L)] = zero16
    for i in range(HR // L):
      hidx[pl.ds(i * L, L)] = iota16 + i * L

    @pl.when(s == 0)
    def _():
      for r in range(0, HR, 8):
        pltpu.sync_copy(zbuf, sh_hist.at[pl.ds(r, 8)])

    ones16 = jnp.ones((L,), jnp.float32)

    def h_chunk(ck, _):
      off = pl.multiple_of(s * EPT + ck * CHUNK, 8)
      pltpu.sync_copy(erow.at[pl.ds(off, CHUNK)], rowck)

      def h_grp(g, _):
        idx = rowck[pl.ds(g * L, L)]
        plsc.addupdate_scatter(hist, [idx >> 7, idx & 127], ones16)
        return 0
      lax.fori_loop(0, CHUNK // L, h_grp, 0)
      return 0
    lax.fori_loop(0, EPT // CHUNK, h_chunk, 0)

    plsc.subcore_barrier()
    pltpu.sync_copy(hist, sh_hist.at[hidx], add=True)
    plsc.subcore_barrier()

    pltpu.sync_copy(sh_hist, hist)

    def rs(r, _):
      for k in range(CK):
        hist[r, pl.ds(k * L, L)] = _rsqrt16(hist[r, pl.ds(k * L, L)])
      return 0
    lax.fori_loop(0, HR, rs, 0)

    lo = w * NPT
    hi = jnp.minimum(lo + NPT, N)
    nr = hi - lo
    ldm = pl.multiple_of(jnp.minimum(lo, N - NPT), 8)
    r0 = lo - ldm

    def s_chunk(ck, cnt):
      pltpu.sync_copy(erow.at[pl.ds(ck * CHUNK, CHUNK)], rowck)
      pltpu.sync_copy(ecol.at[pl.ds(ck * CHUNK, CHUNK)], colck)

      def s_grp(g, cnt):
        rv = rowck[pl.ds(g * L, L)]
        cv = colck[pl.ds(g * L, L)]
        m = (cv >= lo) & (cv < hi)
        dr = plsc.load_gather(hist, [rv >> 7, rv & 127])
        dc = plsc.load_gather(hist, [cv >> 7, cv & 127])
        wv = -(dr * dc)
        cw = (plsc.bitcast(wv, jnp.int32) &
              jnp.int32(-65536)) | (cv - ldm)
        plsc.store_compressed(rowb.at[pl.ds(cnt, L)], rv, mask=m)
        plsc.store_compressed(cwb.at[pl.ds(cnt, L)], cw, mask=m)
        nm = jnp.sum(m.astype(jnp.int32))
        return jnp.minimum(cnt + nm, CAP)
      return lax.fori_loop(0, CHUNK // L, s_grp, cnt)
    cnt = lax.fori_loop(0, E // CHUNK, s_chunk, jnp.int32(0))

    nch = (cnt + (G - 1)) // G

    def batch_body(b, ssq):
      boff = pl.multiple_of(b * N + ldm, 8)
      pltpu.sync_copy(features.at[pl.ds(boff, NPT)], acc)

      def a_chunk(ch, _):
        base = ch * G
        for g in range(G // L):
          gidx[pl.ds(g * L, L)] = rowb[pl.ds(base + g * L, L)] + b * N
        pltpu.sync_copy(features.at[gidx], gbuf)

        def a_grp(g, _):
          cwv = cwb[pl.ds(base + g * L, L)]
          wv = plsc.bitcast(cwv & jnp.int32(-65536), jnp.float32)
          clv = cwv & jnp.int32(0xFFFF)
          for lane in range(L):
            wt = wv[lane]
            cl = clv[lane]
            for k in range(CK):
              plsc.addupdate(acc.at[cl, pl.ds(k * L, L)],
                             wt * gbuf[g * L + lane, pl.ds(k * L, L)])
          return 0
        lax.fori_loop(0, G // L, a_grp, 0)
        return 0
      lax.fori_loop(0, nch, a_chunk, 0)

      def sq_row(r, ssq):
        for k in range(CK):
          v = acc[r0 + r, pl.ds(k * L, L)]
          ssq = ssq + v * v
        return ssq
      return lax.fori_loop(0, nr, sq_row, ssq)

    ssq = lax.fori_loop(0, B, batch_body, zero16)
    sbuf[...] = ssq
    pltpu.sync_copy(sbuf, out.at[pl.ds(pl.multiple_of(w * L, 8), L)])

  return kern


def kernel(features, edge_index):
  B, N, C = features.shape
  E = edge_index.shape[1]
  partials = _make_kernel(B, N, C, E)(features.reshape(B * N, C),
                                      edge_index[0], edge_index[1])
  return jnp.sum(partials) / jnp.float32(B * N * C)


# speedup vs baseline: 18.7755x; 1.1780x over previous
import functools

import jax
import jax.numpy as jnp
from jax import lax
from jax.experimental import pallas as pl
from jax.experimental.pallas import tpu as pltpu
from jax.experimental.pallas import tpu_sc as plsc

L = 16


def _rsqrt16(v):
  i = plsc.bitcast(v, jnp.int32)
  y = plsc.bitcast(jnp.int32(0x5F3759DF) - (i >> 1), jnp.float32)
  for _ in range(3):
    y = y * (1.5 - 0.5 * v * y * y)
  return jnp.where(v > 0.5, y, jnp.zeros_like(y))


def _make_kernel(B, N, C, E, interpret=False):
  NSC = 2
  NT = 16
  NW = NSC * NT
  CK = 8
  assert C == CK * L

  NPT = 8 * (-(-(-(-N // NW)) // 8))
  STG = 64 * (-(-N // (64 * NT)))
  assert STG * NT >= N and N % 8 == 0 and N > NPT
  HR = L * (-(-N // (128 * L)))
  HP = HR * 128
  CHUNK = 2000
  EPT = E // NT
  assert EPT % CHUNK == 0 and E % CHUNK == 0 and CHUNK % L == 0
  assert HR <= 128 and HR % L == 0
  CAP = ((E // NW + 1024) // 128) * 128
  G = 64

  mesh = plsc.VectorSubcoreMesh(core_axis_name="c", subcore_axis_name="s")

  @functools.partial(
      pl.kernel,
      out_type=jax.ShapeDtypeStruct((NW * L,), jnp.float32),
      mesh=mesh,
      interpret=interpret,
      compiler_params=pltpu.CompilerParams(needs_layout_passes=False),
      scratch_types=dict(
          hist=pltpu.VMEM((HR, 128), jnp.float32),
          rowb=pltpu.VMEM((CAP + G,), jnp.int32),
          cwb=pltpu.VMEM((CAP + G,), jnp.int32),
          acc=pltpu.VMEM((NPT, C), jnp.float32),
          gbuf=pltpu.VMEM((G, C), jnp.float32),
          gbuf2=pltpu.VMEM((G, C), jnp.float32),
          gidx=pltpu.VMEM((G,), jnp.int32),
          gidx2=pltpu.VMEM((G,), jnp.int32),
          sem=pltpu.SemaphoreType.DMA,
          sem2=pltpu.SemaphoreType.DMA,
          hidx=pltpu.VMEM((HR,), jnp.int32),
          rowck=pltpu.VMEM((CHUNK,), jnp.int32),
          colck=pltpu.VMEM((CHUNK,), jnp.int32),
          zbuf=pltpu.VMEM((8, C), jnp.float32),
          sbuf=pltpu.VMEM((L,), jnp.float32),
          sh_hist=pltpu.VMEM_SHARED((HR, 128), jnp.float32),
      ),
  )
  def kern(features, erow, ecol, out, hist, rowb, cwb, acc, gbuf, gbuf2,
           gidx, gidx2, sem, sem2, hidx, rowck, colck, zbuf, sbuf, sh_hist):
    s = lax.axis_index("s")
    c = lax.axis_index("c")
    w = c * NT + s
    zero16 = jnp.zeros((L,), jnp.float32)
    iota16 = lax.iota(jnp.int32, L)

    def z_hist(r, _):
      for k in range(CK):
        hist[r, pl.ds(k * L, L)] = zero16
      return 0
    lax.fori_loop(0, HR, z_hist, 0)

    def z_edges(i, _):
      rowb[pl.ds(i * L, L)] = jnp.zeros((L,), jnp.int32)
      cwb[pl.ds(i * L, L)] = jnp.zeros((L,), jnp.int32)
      return 0
    lax.fori_loop(0, (CAP + G) // L, z_edges, 0)

    for k in range(CK):
      for r in range(8):
        zbuf[r, pl.ds(k * L, L)] = zero16
    for i in range(HR // L):
      hidx[pl.ds(i * L, L)] = iota16 + i * L

    @pl.when(s == 0)
    def _():
      for r in range(0, HR, 8):
        pltpu.sync_copy(zbuf, sh_hist.at[pl.ds(r, 8)])

    ones16 = jnp.ones((L,), jnp.float32)

    def h_chunk(ck, _):
      off = pl.multiple_of(s * EPT + ck * CHUNK, 8)
      pltpu.sync_copy(erow.at[pl.ds(off, CHUNK)], rowck)

      def h_grp(g, _):
        idx = rowck[pl.ds(g * L, L)]
        plsc.addupdate_scatter(hist, [idx >> 7, idx & 127], ones16)
        return 0
      lax.fori_loop(0, CHUNK // L, h_grp, 0)
      return 0
    lax.fori_loop(0, EPT // CHUNK, h_chunk, 0)

    plsc.subcore_barrier()
    pltpu.sync_copy(hist, sh_hist.at[hidx], add=True)
    plsc.subcore_barrier()

    pltpu.sync_copy(sh_hist, hist)

    def rs(r, _):
      for k in range(CK):
        hist[r, pl.ds(k * L, L)] = _rsqrt16(hist[r, pl.ds(k * L, L)])
      return 0
    lax.fori_loop(0, HR, rs, 0)

    lo = w * NPT
    hi = jnp.minimum(lo + NPT, N)
    nr = hi - lo
    ldm = pl.multiple_of(jnp.minimum(lo, N - NPT), 8)
    r0 = lo - ldm

    def s_chunk(ck, cnt):
      pltpu.sync_copy(erow.at[pl.ds(ck * CHUNK, CHUNK)], rowck)
      pltpu.sync_copy(ecol.at[pl.ds(ck * CHUNK, CHUNK)], colck)

      def s_grp(g, cnt):
        rv = rowck[pl.ds(g * L, L)]
        cv = colck[pl.ds(g * L, L)]
        m = (cv >= lo) & (cv < hi)
        dr = plsc.load_gather(hist, [rv >> 7, rv & 127])
        dc = plsc.load_gather(hist, [cv >> 7, cv & 127])
        wv = -(dr * dc)
        cw = (plsc.bitcast(wv, jnp.int32) &
              jnp.int32(-65536)) | (cv - ldm)
        plsc.store_compressed(rowb.at[pl.ds(cnt, L)], rv, mask=m)
        plsc.store_compressed(cwb.at[pl.ds(cnt, L)], cw, mask=m)
        nm = jnp.sum(m.astype(jnp.int32))
        return jnp.minimum(cnt + nm, CAP)
      return lax.fori_loop(0, CHUNK // L, s_grp, cnt)
    cnt = lax.fori_loop(0, E // CHUNK, s_chunk, jnp.int32(0))

    npairs = (cnt + (2 * G - 1)) // (2 * G)

    def batch_body(b, ssq):
      boff = pl.multiple_of(b * N + ldm, 8)
      pltpu.sync_copy(features.at[pl.ds(boff, NPT)], acc)

      def fill_idx(buf, base):
        for g in range(G // L):
          buf[pl.ds(g * L, L)] = rowb[pl.ds(base + g * L, L)] + b * N

      def process(buf, base):
        def a_grp(g, _):
          cwv = cwb[pl.ds(base + g * L, L)]
          wv = plsc.bitcast(cwv & jnp.int32(-65536), jnp.float32)
          clv = cwv & jnp.int32(0xFFFF)
          for lane in range(L):
            wt = wv[lane]
            cl = clv[lane]
            for k in range(CK):
              plsc.addupdate(acc.at[cl, pl.ds(k * L, L)],
                             wt * buf[g * L + lane, pl.ds(k * L, L)])
          return 0
        lax.fori_loop(0, G // L, a_grp, 0)

      fill_idx(gidx, 0)
      pltpu.make_async_copy(features.at[gidx], gbuf, sem).start()

      def a_pair(p, _):
        base = p * 2 * G
        fill_idx(gidx2, base + G)
        pltpu.make_async_copy(features.at[gidx2], gbuf2, sem2).start()
        pltpu.make_async_copy(features.at[gidx], gbuf, sem).wait()
        process(gbuf, base)

        @pl.when(p + 1 < npairs)
        def _():
          fill_idx(gidx, base + 2 * G)
          pltpu.make_async_copy(features.at[gidx], gbuf, sem).start()
        pltpu.make_async_copy(features.at[gidx2], gbuf2, sem2).wait()
        process(gbuf2, base + G)
        return 0
      lax.fori_loop(0, npairs, a_pair, 0)

      def sq_row(r, ssq):
        for k in range(CK):
          v = acc[r0 + r, pl.ds(k * L, L)]
          ssq = ssq + v * v
        return ssq
      return lax.fori_loop(0, nr, sq_row, ssq)

    ssq = lax.fori_loop(0, B, batch_body, zero16)
    sbuf[...] = ssq
    pltpu.sync_copy(sbuf, out.at[pl.ds(pl.multiple_of(w * L, 8), L)])

  return kern


def kernel(features, edge_index):
  B, N, C = features.shape
  E = edge_index.shape[1]
  partials = _make_kernel(B, N, C, E)(features.reshape(B * N, C),
                                      edge_index[0], edge_index[1])
  return jnp.sum(partials) / jnp.float32(B * N * C)
